# Initial kernel scaffold; baseline (speedup 1.0000x reference)
import functools

import jax
import jax.numpy as jnp
from jax import lax
from jax.experimental import pallas as pl
from jax.experimental.pallas import tpu as pltpu
from jax.experimental.pallas import tpu_sc as plsc

N = 10000
E = 320000
NG = 16
NC = 2
NS = 16
L = 16

EPT_DEN = E // (NC * NS)
EPT_AGG = E // NS
K = 80


def _leaky(x, slope):
    return jnp.where(x >= 0, x, slope * x)


def _tc_mm(h_in, W, a_s, a_d, bias, stacked):
    din, c = W.shape
    half = c // 2
    bn = 2000
    grid = (N // bn,)
    a_s2 = a_s.reshape(1, c)
    a_d2 = a_d.reshape(1, c)

    def body(*refs):
        if stacked:
            h_ref, w_ref, as_ref, ad_ref, b_ref, xw_ref, asr_ref, adr_ref = refs
            h = jnp.concatenate([h_ref[0], h_ref[1]], axis=1)
            h = _leaky(h + b_ref[...], 0.01)
        else:
            h_ref, w_ref, as_ref, ad_ref, xw_ref, asr_ref, adr_ref = refs
            h = h_ref[...]
        xw = jnp.dot(h, w_ref[...], preferred_element_type=jnp.float32)
        xw_ref[0] = xw[:, :half]
        xw_ref[1] = xw[:, half:]
        asr_ref[...] = jnp.sum(xw * as_ref[...], axis=1, keepdims=True)
        adr_ref[...] = jnp.sum(xw * ad_ref[...], axis=1, keepdims=True)

    if stacked:
        in_specs = [
            pl.BlockSpec((2, bn, din // 2), lambda i: (0, i, 0)),
            pl.BlockSpec((din, c), lambda i: (0, 0)),
            pl.BlockSpec((1, c), lambda i: (0, 0)),
            pl.BlockSpec((1, c), lambda i: (0, 0)),
            pl.BlockSpec((1, din), lambda i: (0, 0)),
        ]
    else:
        in_specs = [
            pl.BlockSpec((bn, din), lambda i: (i, 0)),
            pl.BlockSpec((din, c), lambda i: (0, 0)),
            pl.BlockSpec((1, c), lambda i: (0, 0)),
            pl.BlockSpec((1, c), lambda i: (0, 0)),
        ]
    out_specs = [
        pl.BlockSpec((2, bn, half), lambda i: (0, i, 0)),
        pl.BlockSpec((bn, 1), lambda i: (i, 0)),
        pl.BlockSpec((bn, 1), lambda i: (i, 0)),
    ]
    out_shape = [
        jax.ShapeDtypeStruct((2, N, half), jnp.float32),
        jax.ShapeDtypeStruct((N, 1), jnp.float32),
        jax.ShapeDtypeStruct((N, 1), jnp.float32),
    ]
    args = (h_in, W, a_s2, a_d2) + ((bias.reshape(1, din),) if stacked else ())
    return pl.pallas_call(
        body, grid=grid, in_specs=in_specs, out_specs=out_specs,
        out_shape=out_shape)(*args)


def _edge_denom(src, dst, asrc, adst, m8):
    mesh = plsc.VectorSubcoreMesh(core_axis_name="c", subcore_axis_name="s")

    @functools.partial(
        pl.kernel,
        out_type=jax.ShapeDtypeStruct((NC, N), jnp.float32),
        mesh=mesh,
        scratch_types=[
            pltpu.VMEM((N,), jnp.float32),
            pltpu.VMEM((N,), jnp.float32),
            pltpu.VMEM((8,), jnp.float32),
            pltpu.VMEM((EPT_DEN,), jnp.int32),
            pltpu.VMEM((EPT_DEN,), jnp.int32),
            pltpu.VMEM((K,), jnp.float32),
            pltpu.VMEM((K,), jnp.int32),
            pltpu.VMEM((1008,), jnp.float32),
            pltpu.VMEM_SHARED((N,), jnp.float32),
        ],
    )
    def k(src_h, dst_h, asrc_h, adst_h, m_h, den_h,
          asrc_v, adst_v, m_v, sbuf, dbuf, eebuf, didx, zbuf, den_sp):
        c = lax.axis_index("c")
        s = lax.axis_index("s")
        pltpu.sync_copy(asrc_h, asrc_v)
        pltpu.sync_copy(adst_h, adst_v)
        pltpu.sync_copy(m_h, m_v)
        base = (c * NS + s) * EPT_DEN
        pltpu.sync_copy(src_h.at[pl.ds(base, EPT_DEN)], sbuf)
        pltpu.sync_copy(dst_h.at[pl.ds(base, EPT_DEN)], dbuf)

        def zz(i, _):
            zbuf[pl.ds(i * L, L)] = jnp.zeros((L,), jnp.float32)
            return 0
        lax.fori_loop(0, 63, zz, 0)

        @pl.when(s < NS - 1)
        def _():
            pltpu.sync_copy(zbuf.at[pl.ds(0, 640)],
                            den_sp.at[pl.ds(s * 640, 640)])

        @pl.when(s == NS - 1)
        def _():
            pltpu.sync_copy(zbuf.at[pl.ds(0, 400)],
                            den_sp.at[pl.ds(9600, 400)])

        plsc.subcore_barrier()

        m = m_v[0]

        def chunk(ch, _):
            for j in range(K // L):
                off = ch * K + j * L
                sv = sbuf[pl.ds(off, L)]
                dv = dbuf[pl.ds(off, L)]
                val = (plsc.load_gather(asrc_v, [sv])
                       + plsc.load_gather(adst_v, [dv]))
                val = _leaky(val, 0.2)
                eebuf[pl.ds(j * L, L)] = jnp.exp(val - m)
                didx[pl.ds(j * L, L)] = dv
            pltpu.sync_copy(eebuf, den_sp.at[didx], add=True)
            return 0
        lax.fori_loop(0, EPT_DEN // K, chunk, 0)

        plsc.subcore_barrier()

        @pl.when(s < 10)
        def _():
            pltpu.sync_copy(den_sp.at[pl.ds(s * 1000, 1000)],
                            zbuf.at[pl.ds(0, 1000)])
            pltpu.sync_copy(zbuf.at[pl.ds(0, 1000)],
                            den_h.at[c, pl.ds(s * 1000, 1000)])

    return k(src, dst, asrc, adst, m8)


def _edge_agg(src, dst, asrc, adst, m8, den, xw_flat, cs):
    mesh = plsc.VectorSubcoreMesh(core_axis_name="c", subcore_axis_name="s")
    rpt = N // NS
    scratch = [
        pltpu.VMEM((N,), jnp.float32),
        pltpu.VMEM((N,), jnp.float32),
        pltpu.VMEM((N,), jnp.float32),
        pltpu.VMEM((N,), jnp.float32),
        pltpu.VMEM((8,), jnp.float32),
        pltpu.VMEM((EPT_AGG,), jnp.int32),
        pltpu.VMEM((EPT_AGG,), jnp.int32),
        pltpu.VMEM((K,), jnp.float32),
        pltpu.VMEM((K,), jnp.int32),
        pltpu.VMEM((K,), jnp.int32),
        pltpu.VMEM((K, cs), jnp.float32),
        pltpu.VMEM((125, cs), jnp.float32),
        pltpu.VMEM_SHARED((N, cs), jnp.float32),
        pltpu.SemaphoreType.DMA,
    ]

    @functools.partial(
        pl.kernel,
        out_type=jax.ShapeDtypeStruct((NC, N, cs), jnp.float32),
        mesh=mesh,
        scratch_types=scratch,
    )
    def k(src_h, dst_h, asrc_h, adst_h, m_h, den_h, xw_h, out_h,
          asrc_v, adst_v, d0, d1, m_v, sbuf, dbuf, albuf, gidx, didx,
          rows, drainbuf, acc, sem):
        c = lax.axis_index("c")
        s = lax.axis_index("s")
        pltpu.sync_copy(asrc_h, asrc_v)
        pltpu.sync_copy(adst_h, adst_v)
        pltpu.sync_copy(den_h.at[0], d0)
        pltpu.sync_copy(den_h.at[1], d1)
        pltpu.sync_copy(m_h, m_v)
        base = s * EPT_AGG
        pltpu.sync_copy(src_h.at[pl.ds(base, EPT_AGG)], sbuf)
        pltpu.sync_copy(dst_h.at[pl.ds(base, EPT_AGG)], dbuf)

        def dtot(i, _):
            d0[pl.ds(i * L, L)] = d0[pl.ds(i * L, L)] + d1[pl.ds(i * L, L)]
            return 0
        lax.fori_loop(0, N // L, dtot, 0)

        def zz(i, _):
            for u in range(cs // L):
                drainbuf[i, pl.ds(u * L, L)] = jnp.zeros((L,), jnp.float32)
            return 0
        lax.fori_loop(0, 125, zz, 0)
        for t in range(5):
            pltpu.sync_copy(drainbuf, acc.at[pl.ds(s * rpt + t * 125, 125)])
        plsc.subcore_barrier()

        m = m_v[0]

        def chunk(ch, _):
            for j in range(K // L):
                off = ch * K + j * L
                sv = sbuf[pl.ds(off, L)]
                dv = dbuf[pl.ds(off, L)]
                val = (plsc.load_gather(asrc_v, [sv])
                       + plsc.load_gather(adst_v, [dv]))
                val = _leaky(val, 0.2)
                ee = jnp.exp(val - m)
                dn = plsc.load_gather(d0, [dv])
                albuf[pl.ds(j * L, L)] = ee / (dn + 1e-16)
                gidx[pl.ds(j * L, L)] = sv + c * N
                didx[pl.ds(j * L, L)] = dv
            pltpu.async_copy(xw_h.at[gidx], rows, sem).wait()

            def scale(r, _):
                a = albuf[r]
                for u in range(cs // L):
                    rows[r, pl.ds(u * L, L)] = rows[r, pl.ds(u * L, L)] * a
                return 0
            lax.fori_loop(0, K, scale, 0)
            pltpu.sync_copy(rows, acc.at[didx], add=True)
            return 0
        lax.fori_loop(0, EPT_AGG // K, chunk, 0)

        plsc.subcore_barrier()
        for t in range(5):
            r0 = s * rpt + t * 125
            pltpu.sync_copy(acc.at[pl.ds(r0, 125)], drainbuf)
            pltpu.sync_copy(drainbuf, out_h.at[c, pl.ds(r0, 125)])

    return k(src, dst, asrc, adst, m8, den, xw_flat)


def _layer(h_in, src, dst, W, a_s, a_d, bias, stacked):
    xw, asr, adr = _tc_mm(h_in, W, a_s, a_d, bias, stacked)
    half = W.shape[1] // 2
    asrf = asr.reshape(N)
    adrf = adr.reshape(N)
    m = jnp.max(asrf) + jnp.max(adrf)
    m8 = jnp.full((8,), m, jnp.float32)
    den = _edge_denom(src, dst, asrf, adrf, m8)
    return _edge_agg(src, dst, asrf, adrf, m8, den,
                     xw.reshape(NC * N, half), half)


def kernel(x, edge_index, batch, W1, a1s, a1d, b1, W2, a2s, a2d, b2,
           W3, a3s, a3d, b3):
    src = edge_index[0]
    dst = edge_index[1]
    raw1 = _layer(x, src, dst, W1, a1s, a1d, None, False)
    raw2 = _layer(raw1, src, dst, W2, a2s, a2d, b1, True)
    raw3 = _layer(raw2, src, dst, W3, a3s, a3d, b2, True)
    h3 = jnp.concatenate([raw3[0], raw3[1]], axis=1) + b3

    bb = jnp.concatenate([jnp.zeros((1,), batch.dtype), batch])
    diff = (bb[1:] - bb[:-1]).at[0].set(1)
    idx = jnp.flatnonzero(diff.astype(bool), size=NG)
    return h3[idx, :]


# speedup vs baseline: 17.3180x; 17.3180x over previous
import functools

import jax
import jax.numpy as jnp
from jax import lax
from jax.experimental import pallas as pl
from jax.experimental.pallas import tpu as pltpu
from jax.experimental.pallas import tpu_sc as plsc

N = 10000
E = 320000
NG = 16
NC = 2
NS = 16
L = 16

EPT_DEN = E // (NC * NS)
EPT_AGG = E // NS
K = 80


def _leaky(x, slope):
    return jnp.where(x >= 0, x, slope * x)


def _tc_mm(h_in, W, a_s, a_d, bias, stacked, split_out=True):
    din, c = W.shape
    half = c // 2
    bn = 2000
    grid = (N // bn,)
    a_s2 = a_s.reshape(1, c)
    a_d2 = a_d.reshape(1, c)

    def body(*refs):
        if stacked:
            h_ref, w_ref, as_ref, ad_ref, b_ref, xw_ref, asr_ref, adr_ref = refs
            h = jnp.concatenate([h_ref[0], h_ref[1]], axis=1)
            h = _leaky(h + b_ref[...], 0.01)
        else:
            h_ref, w_ref, as_ref, ad_ref, xw_ref, asr_ref, adr_ref = refs
            h = h_ref[...]
        xw = jnp.dot(h, w_ref[...], preferred_element_type=jnp.float32)
        if split_out:
            xw_ref[0] = xw[:, :half]
            xw_ref[1] = xw[:, half:]
        else:
            xw_ref[...] = xw
        asr_ref[...] = jnp.sum(xw * as_ref[...], axis=1, keepdims=True)
        adr_ref[...] = jnp.sum(xw * ad_ref[...], axis=1, keepdims=True)

    if stacked:
        in_specs = [
            pl.BlockSpec((2, bn, din // 2), lambda i: (0, i, 0)),
            pl.BlockSpec((din, c), lambda i: (0, 0)),
            pl.BlockSpec((1, c), lambda i: (0, 0)),
            pl.BlockSpec((1, c), lambda i: (0, 0)),
            pl.BlockSpec((1, din), lambda i: (0, 0)),
        ]
    else:
        in_specs = [
            pl.BlockSpec((bn, din), lambda i: (i, 0)),
            pl.BlockSpec((din, c), lambda i: (0, 0)),
            pl.BlockSpec((1, c), lambda i: (0, 0)),
            pl.BlockSpec((1, c), lambda i: (0, 0)),
        ]
    if split_out:
        xw_spec = pl.BlockSpec((2, bn, half), lambda i: (0, i, 0))
        xw_shape = jax.ShapeDtypeStruct((2, N, half), jnp.float32)
    else:
        xw_spec = pl.BlockSpec((bn, c), lambda i: (i, 0))
        xw_shape = jax.ShapeDtypeStruct((N, c), jnp.float32)
    out_specs = [
        xw_spec,
        pl.BlockSpec((bn, 1), lambda i: (i, 0)),
        pl.BlockSpec((bn, 1), lambda i: (i, 0)),
    ]
    out_shape = [
        xw_shape,
        jax.ShapeDtypeStruct((N, 1), jnp.float32),
        jax.ShapeDtypeStruct((N, 1), jnp.float32),
    ]
    args = (h_in, W, a_s2, a_d2) + ((bias.reshape(1, din),) if stacked else ())
    return pl.pallas_call(
        body, grid=grid, in_specs=in_specs, out_specs=out_specs,
        out_shape=out_shape)(*args)


def _edge_denom(src, dst, asrc, adst, m8):
    mesh = plsc.VectorSubcoreMesh(core_axis_name="c", subcore_axis_name="s")

    @functools.partial(
        pl.kernel,
        out_type=jax.ShapeDtypeStruct((NC * N,), jnp.float32),
        mesh=mesh,
        compiler_params=pltpu.CompilerParams(needs_layout_passes=False),
        scratch_types=[
            pltpu.VMEM((N,), jnp.float32),
            pltpu.VMEM((N,), jnp.float32),
            pltpu.VMEM((L,), jnp.float32),
            pltpu.VMEM((EPT_DEN,), jnp.int32),
            pltpu.VMEM((EPT_DEN,), jnp.int32),
            pltpu.VMEM((K,), jnp.float32),
            pltpu.VMEM((K,), jnp.int32),
            pltpu.VMEM((1008,), jnp.float32),
            pltpu.VMEM_SHARED((N,), jnp.float32),
        ],
    )
    def k(src_h, dst_h, asrc_h, adst_h, m_h, den_h,
          asrc_v, adst_v, m_v, sbuf, dbuf, eebuf, didx, zbuf, den_sp):
        c = lax.axis_index("c")
        s = lax.axis_index("s")
        pltpu.sync_copy(asrc_h, asrc_v)
        pltpu.sync_copy(adst_h, adst_v)
        pltpu.sync_copy(m_h, m_v)
        base = (c * NS + s) * EPT_DEN
        pltpu.sync_copy(src_h.at[pl.ds(base, EPT_DEN)], sbuf)
        pltpu.sync_copy(dst_h.at[pl.ds(base, EPT_DEN)], dbuf)

        def zz(i, _):
            zbuf[pl.ds(i * L, L)] = jnp.zeros((L,), jnp.float32)
            return 0
        lax.fori_loop(0, 63, zz, 0)

        @pl.when(s < NS - 1)
        def _():
            pltpu.sync_copy(zbuf.at[pl.ds(0, 640)],
                            den_sp.at[pl.ds(s * 640, 640)])

        @pl.when(s == NS - 1)
        def _():
            pltpu.sync_copy(zbuf.at[pl.ds(0, 400)],
                            den_sp.at[pl.ds(9600, 400)])

        plsc.subcore_barrier()

        m = m_v[...]

        def chunk(ch, _):
            for j in range(K // L):
                off = ch * K + j * L
                sv = sbuf[pl.ds(off, L)]
                dv = dbuf[pl.ds(off, L)]
                val = (plsc.load_gather(asrc_v, [sv])
                       + plsc.load_gather(adst_v, [dv]))
                val = _leaky(val, 0.2)
                eebuf[pl.ds(j * L, L)] = jnp.exp(val - m)
                didx[pl.ds(j * L, L)] = dv
            pltpu.sync_copy(eebuf, den_sp.at[didx], add=True)
            return 0
        lax.fori_loop(0, EPT_DEN // K, chunk, 0)

        plsc.subcore_barrier()

        @pl.when(s < 10)
        def _():
            pltpu.sync_copy(den_sp.at[pl.ds(s * 1000, 1000)],
                            zbuf.at[pl.ds(0, 1000)])
            pltpu.sync_copy(zbuf.at[pl.ds(0, 1000)],
                            den_h.at[pl.ds(c * N + s * 1000, 1000)])

    return k(src, dst, asrc, adst, m8)


def _edge_agg(src, dst, asrc, adst, m8, den, xw_flat, cs, feat_split=True):
    mesh = plsc.VectorSubcoreMesh(core_axis_name="c", subcore_axis_name="s")
    STG = 2000
    DCH = 2000
    scratch = [
        pltpu.VMEM((N,), jnp.float32),
        pltpu.VMEM((N,), jnp.float32),
        pltpu.VMEM((N,), jnp.float32),
        pltpu.VMEM((DCH,), jnp.float32),
        pltpu.VMEM((L,), jnp.float32),
        pltpu.VMEM((STG,), jnp.int32),
        pltpu.VMEM((STG,), jnp.int32),
        pltpu.VMEM((K + L,), jnp.float32),
        pltpu.VMEM((K,), jnp.int32),
        pltpu.VMEM((K,), jnp.int32),
        pltpu.VMEM((K, cs), jnp.float32),
        pltpu.VMEM_SHARED((N, cs), jnp.float32),
        pltpu.SemaphoreType.DMA,
    ]

    @functools.partial(
        pl.kernel,
        out_type=jax.ShapeDtypeStruct((NC, N, cs), jnp.float32),
        mesh=mesh,
        compiler_params=pltpu.CompilerParams(needs_layout_passes=False),
        scratch_types=scratch,
    )
    def k(src_h, dst_h, asrc_h, adst_h, m_h, den_h, xw_h, out_h,
          asrc_v, adst_v, d0, d1, m_v, sbuf, dbuf, albuf, gidx, didx,
          rows, acc, sem):
        c = lax.axis_index("c")
        s = lax.axis_index("s")
        pltpu.sync_copy(asrc_h, asrc_v)
        pltpu.sync_copy(adst_h, adst_v)
        pltpu.sync_copy(den_h.at[pl.ds(0, N)], d0)
        pltpu.sync_copy(m_h, m_v)

        def dtot(jj, _):
            pltpu.sync_copy(den_h.at[pl.ds(N + jj * DCH, DCH)], d1)

            def dadd(i, _):
                o = jj * DCH + i * L
                d0[pl.ds(o, L)] = d0[pl.ds(o, L)] + d1[pl.ds(i * L, L)]
                return 0
            lax.fori_loop(0, DCH // L, dadd, 0)
            return 0
        lax.fori_loop(0, N // DCH, dtot, 0)

        def zz(i, _):
            for u in range(cs // L):
                rows[i, pl.ds(u * L, L)] = jnp.zeros((L,), jnp.float32)
            return 0
        lax.fori_loop(0, 40, zz, 0)
        nch = jnp.where(s < NS - 1, 16, 10)

        def zc(t, _):
            pltpu.sync_copy(rows.at[pl.ds(0, 40)],
                            acc.at[pl.ds(s * 640 + t * 40, 40)])
            return 0
        lax.fori_loop(0, nch, zc, 0)
        plsc.subcore_barrier()

        m = m_v[...]

        ept = EPT_AGG if feat_split else EPT_DEN

        def stage(st, _):
            if feat_split:
                b2 = s * EPT_AGG + st * STG
            else:
                b2 = (c * NS + s) * EPT_DEN + st * STG
            pltpu.sync_copy(src_h.at[pl.ds(b2, STG)], sbuf)
            pltpu.sync_copy(dst_h.at[pl.ds(b2, STG)], dbuf)

            def chunk(ch, _):
                for j in range(K // L):
                    off = ch * K + j * L
                    sv = sbuf[pl.ds(off, L)]
                    dv = dbuf[pl.ds(off, L)]
                    val = (plsc.load_gather(asrc_v, [sv])
                           + plsc.load_gather(adst_v, [dv]))
                    val = _leaky(val, 0.2)
                    ee = jnp.exp(val - m)
                    dn = plsc.load_gather(d0, [dv])
                    albuf[pl.ds(j * L, L)] = ee / (dn + 1e-16)
                    if feat_split:
                        gidx[pl.ds(j * L, L)] = sv + c * N
                    else:
                        gidx[pl.ds(j * L, L)] = sv
                    didx[pl.ds(j * L, L)] = dv
                pltpu.async_copy(xw_h.at[gidx], rows, sem).wait()

                def scale(r, _):
                    a = albuf[pl.ds(r, L)][0]
                    for u in range(cs // L):
                        rows[r, pl.ds(u * L, L)] = rows[r, pl.ds(u * L, L)] * a
                    return 0
                lax.fori_loop(0, K, scale, 0)
                pltpu.sync_copy(rows, acc.at[didx], add=True)
                return 0
            lax.fori_loop(0, STG // K, chunk, 0)
            return 0
        lax.fori_loop(0, ept // STG, stage, 0)

        plsc.subcore_barrier()

        def dr(t, _):
            r0 = s * 640 + t * 40
            pltpu.sync_copy(acc.at[pl.ds(r0, 40)], rows.at[pl.ds(0, 40)])
            pltpu.sync_copy(rows.at[pl.ds(0, 40)], out_h.at[c, pl.ds(r0, 40)])
            return 0
        lax.fori_loop(0, nch, dr, 0)

    return k(src, dst, asrc, adst, m8, den, xw_flat)


def _layer(h_in, src, dst, W, a_s, a_d, bias, stacked, feat_split=True):
    c = W.shape[1]
    xw, asr, adr = _tc_mm(h_in, W, a_s, a_d, bias, stacked,
                          split_out=feat_split)
    asrf = asr.reshape(N)
    adrf = adr.reshape(N)
    m = jnp.max(asrf) + jnp.max(adrf)
    m16 = jnp.full((16,), m, jnp.float32)
    den = _edge_denom(src, dst, asrf, adrf, m16)
    if feat_split:
        return _edge_agg(src, dst, asrf, adrf, m16, den,
                         xw.reshape(NC * N, c // 2), c // 2)
    return _edge_agg(src, dst, asrf, adrf, m16, den, xw, c,
                     feat_split=False)


def kernel(x, edge_index, batch, W1, a1s, a1d, b1, W2, a2s, a2d, b2,
           W3, a3s, a3d, b3):
    src = edge_index[0]
    dst = edge_index[1]
    raw1 = _layer(x, src, dst, W1, a1s, a1d, None, False)
    raw2 = _layer(raw1, src, dst, W2, a2s, a2d, b1, True)
    raw3 = _layer(raw2, src, dst, W3, a3s, a3d, b2, True, feat_split=False)

    bb = jnp.concatenate([jnp.zeros((1,), batch.dtype), batch])
    diff = (bb[1:] - bb[:-1]).at[0].set(1)
    idx = jnp.flatnonzero(diff.astype(bool), size=NG)
    return raw3[0][idx, :] + raw3[1][idx, :] + b3
